# Initial kernel scaffold; baseline (speedup 1.0000x reference)
import jax
import jax.numpy as jnp
from jax.experimental import pallas as pl


def kernel(input_ids, token_type_ids, token_embedding, position_table, type_table, ln_gamma, ln_beta):
    raise NotImplementedError("write your pallas kernel here")


# speedup vs baseline: 2.3948x; 2.3948x over previous
import functools

import jax
import jax.numpy as jnp
from jax import lax
from jax.experimental import pallas as pl
from jax.experimental.pallas import tpu as pltpu
from jax.experimental.pallas import tpu_sc as plsc

VOCAB = 30000
TYPE_VOCAB = 300
HIDDEN = 768
MAX_POS = 200
B = 1024
L = 200

N_TOK = B * L
NC, NS = 2, 16
NW = NC * NS
TOK_PER_W = N_TOK // NW
CHUNK = 64
N_CHUNKS = TOK_PER_W // CHUNK


def _sc_gather_body(table_hbm, ids_hbm, out_hbm, ids_v, buf0, buf1, sem0, sem1):
    wid = lax.axis_index("s") * NC + lax.axis_index("c")
    base = wid * TOK_PER_W
    pltpu.sync_copy(ids_hbm.at[pl.ds(base, TOK_PER_W)], ids_v)

    def start_gather(c, buf, sem):
        pltpu.make_async_copy(
            table_hbm.at[ids_v.at[pl.ds(c * CHUNK, CHUNK)]], buf, sem
        ).start()

    def wait_gather(buf, sem):
        pltpu.make_async_copy(table_hbm.at[ids_v.at[pl.ds(0, CHUNK)]], buf, sem).wait()

    start_gather(0, buf0, sem0)
    start_gather(1, buf1, sem1)

    def body(i, carry):
        c0 = 2 * i
        wait_gather(buf0, sem0)
        pltpu.sync_copy(buf0, out_hbm.at[pl.ds(base + c0 * CHUNK, CHUNK)])

        @pl.when(c0 + 2 < N_CHUNKS)
        def _():
            start_gather(c0 + 2, buf0, sem0)

        wait_gather(buf1, sem1)
        pltpu.sync_copy(buf1, out_hbm.at[pl.ds(base + (c0 + 1) * CHUNK, CHUNK)])

        @pl.when(c0 + 3 < N_CHUNKS)
        def _():
            start_gather(c0 + 3, buf1, sem1)

        return carry

    lax.fori_loop(0, N_CHUNKS // 2, body, 0)


@functools.partial(
    pl.kernel,
    out_type=jax.ShapeDtypeStruct((N_TOK, HIDDEN), jnp.float32),
    mesh=plsc.VectorSubcoreMesh(core_axis_name="c", subcore_axis_name="s"),
    scratch_types=[
        pltpu.VMEM((TOK_PER_W,), jnp.int32),
        pltpu.VMEM((CHUNK, HIDDEN), jnp.float32),
        pltpu.VMEM((CHUNK, HIDDEN), jnp.float32),
        pltpu.SemaphoreType.DMA,
        pltpu.SemaphoreType.DMA,
    ],
)
def _sc_gather(table_hbm, ids_hbm, out_hbm, ids_v, buf0, buf1, sem0, sem1):
    _sc_gather_body(table_hbm, ids_hbm, out_hbm, ids_v, buf0, buf1, sem0, sem1)


BB = 8


def _tc_fuse_body(tids_ref, gat_ref, type_ref, pos_ref, gamma_ref, beta_ref, out_ref):
    x = gat_ref[...]
    tids = tids_ref[...]
    ttab = type_ref[...]

    onehot = (
        tids == lax.broadcasted_iota(jnp.int32, (BB, L, TYPE_VOCAB), 2)
    ).astype(jnp.float32).reshape(BB * L, TYPE_VOCAB)
    typ = jnp.dot(onehot, ttab, preferred_element_type=jnp.float32)
    x = x + typ.reshape(BB, L, HIDDEN) + pos_ref[...][None, :, :]

    mean = jnp.mean(x, axis=-1, keepdims=True)
    xc = x - mean
    var = jnp.mean(xc * xc, axis=-1, keepdims=True)
    inv = lax.rsqrt(var + 1e-12)
    out_ref[...] = xc * inv * gamma_ref[...] + beta_ref[...]


def _tc_fuse(tids, gathered, type_table, position_table, ln_gamma, ln_beta):
    grid = (B // BB,)
    return pl.pallas_call(
        _tc_fuse_body,
        grid=grid,
        in_specs=[
            pl.BlockSpec((BB, L, 1), lambda i: (i, 0, 0)),
            pl.BlockSpec((BB, L, HIDDEN), lambda i: (i, 0, 0)),
            pl.BlockSpec((TYPE_VOCAB, HIDDEN), lambda i: (0, 0)),
            pl.BlockSpec((MAX_POS, HIDDEN), lambda i: (0, 0)),
            pl.BlockSpec((HIDDEN,), lambda i: (0,)),
            pl.BlockSpec((HIDDEN,), lambda i: (0,)),
        ],
        out_specs=pl.BlockSpec((BB, L, HIDDEN), lambda i: (i, 0, 0)),
        out_shape=jax.ShapeDtypeStruct((B, L, HIDDEN), jnp.float32),
    )(tids, gathered, type_table, position_table, ln_gamma, ln_beta)


@jax.jit
def kernel(input_ids, token_type_ids, token_embedding, position_table, type_table,
           ln_gamma, ln_beta):
    ids_flat = input_ids.reshape(-1).astype(jnp.int32)
    gathered = _sc_gather(token_embedding, ids_flat)
    gathered = gathered.reshape(B, L, HIDDEN)
    tids3 = token_type_ids.astype(jnp.int32).reshape(B, L, 1)
    return _tc_fuse(tids3, gathered, type_table, position_table, ln_gamma, ln_beta)
